# Initial kernel scaffold; baseline (speedup 1.0000x reference)
import jax
import jax.numpy as jnp
from jax.experimental import pallas as pl


def kernel(x, y, t2v_w, t2v_b, local_table, vt_w, vt_b, space_table, given_table):
    raise NotImplementedError("write your pallas kernel here")


# speedup vs baseline: 5.7043x; 5.7043x over previous
import jax
import jax.numpy as jnp
from jax import lax
from jax.experimental import pallas as pl
from jax.experimental.pallas import tpu as pltpu

_B, _L, _DY, _DX, _DM = 8, 512, 32, 6, 128
_K = 6


def _tc_body(x_ref, y4_ref, t2vw_ref, t2vb_ref, local_ref, vtw_ref, vtb_ref,
             space_ref, given_ref, out1_ref, out2_ref, out3_ref, base_ref):
    d = pl.program_id(1)

    @pl.when(d == 0)
    def _compute_base():
        xs = x_ref[0]
        xs = jnp.where(jnp.isnan(xs), 0.0, xs)
        acc = local_ref[...] + vtb_ref[...] + given_ref[1:2, :]
        for i in range(_DX):
            aff = xs[:, i:i + 1] * t2vw_ref[i:i + 1, :] + t2vb_ref[i:i + 1, :]
            col = lax.broadcasted_iota(jnp.int32, aff.shape, 1)
            te_i = jnp.where(col == 0, aff, jnp.sin(aff))
            acc = acc + jnp.dot(te_i, vtw_ref[i * _K:(i + 1) * _K, :],
                                preferred_element_type=jnp.float32)
        base_ref[...] = acc

    yc = y4_ref[0, 0]
    nan = jnp.isnan(yc)
    ycc = jnp.where(nan, 0.0, yc)
    w_y = vtw_ref[_DX * _K:_DX * _K + 1, :]
    delta = given_ref[0:1, :] - given_ref[1:2, :]
    out1_ref[0] = (base_ref[...] + ycc * w_y
                   + jnp.where(nan, 1.0, 0.0) * delta)
    out2_ref[0] = jnp.broadcast_to(space_ref[0], (_L, _DM))
    out3_ref[0] = jnp.full((1, _L), d, dtype=jnp.int32)


def kernel(x, y, t2v_w, t2v_b, local_table, vt_w, vt_b, space_table,
           given_table):
    batch, length, dy = y.shape
    x3 = x.reshape(batch, length, _DX)
    y4 = jnp.transpose(y, (0, 2, 1)).reshape(batch, dy, length, 1)
    vtb2 = vt_b.reshape(1, _DM)
    space3 = space_table.reshape(dy, 1, _DM)

    grid = (batch, dy)
    out1, out2, out3 = pl.pallas_call(
        _tc_body,
        grid=grid,
        in_specs=[
            pl.BlockSpec((1, length, _DX), lambda b, d: (b, 0, 0)),
            pl.BlockSpec((1, 1, length, 1), lambda b, d: (b, d, 0, 0)),
            pl.BlockSpec((_DX, _K), lambda b, d: (0, 0)),
            pl.BlockSpec((_DX, _K), lambda b, d: (0, 0)),
            pl.BlockSpec((length, _DM), lambda b, d: (0, 0)),
            pl.BlockSpec((_DX * _K + 1, _DM), lambda b, d: (0, 0)),
            pl.BlockSpec((1, _DM), lambda b, d: (0, 0)),
            pl.BlockSpec((1, 1, _DM), lambda b, d: (d, 0, 0)),
            pl.BlockSpec((2, _DM), lambda b, d: (0, 0)),
        ],
        out_specs=[
            pl.BlockSpec((1, length, _DM), lambda b, d: (b, d, 0)),
            pl.BlockSpec((1, length, _DM), lambda b, d: (b, d, 0)),
            pl.BlockSpec((1, 1, length), lambda b, d: (b * dy + d, 0, 0)),
        ],
        out_shape=[
            jax.ShapeDtypeStruct((batch, dy * length, _DM), jnp.float32),
            jax.ShapeDtypeStruct((batch, dy * length, _DM), jnp.float32),
            jax.ShapeDtypeStruct((batch * dy, 1, length), jnp.int32),
        ],
        scratch_shapes=[pltpu.VMEM((length, _DM), jnp.float32)],
        compiler_params=pltpu.CompilerParams(
            dimension_semantics=("arbitrary", "arbitrary")),
    )(x3, y4, t2v_w, t2v_b, local_table[:length], vt_w, vtb2, space3,
      given_table)

    return (out1, out2, out3.reshape(batch, dy * length))


# speedup vs baseline: 9.7994x; 1.7179x over previous
import jax
import jax.numpy as jnp
from jax import lax
from jax.experimental import pallas as pl
from jax.experimental.pallas import tpu as pltpu

_B, _L, _DY, _DX, _DM = 8, 512, 32, 6, 128
_K = 6


_DC = 8


def _tc_body(x_ref, y4_ref, t2vw_ref, t2vb_ref, local_ref, vtw_ref, vtb_ref,
             space_ref, given_ref, out1_ref, out2_ref, out3_ref, base_ref):
    c = pl.program_id(1)

    @pl.when(c == 0)
    def _compute_base():
        xs = x_ref[0]
        xs = jnp.where(jnp.isnan(xs), 0.0, xs)
        acc = local_ref[...] + vtb_ref[...] + given_ref[1:2, :]
        for i in range(_DX):
            aff = xs[:, i:i + 1] * t2vw_ref[i:i + 1, :] + t2vb_ref[i:i + 1, :]
            col = lax.broadcasted_iota(jnp.int32, aff.shape, 1)
            te_i = jnp.where(col == 0, aff, jnp.sin(aff))
            acc = acc + jnp.dot(te_i, vtw_ref[i * _K:(i + 1) * _K, :],
                                preferred_element_type=jnp.float32)
        base_ref[...] = acc

    w_y = vtw_ref[_DX * _K:_DX * _K + 1, :]
    delta = given_ref[0:1, :] - given_ref[1:2, :]
    base = base_ref[...]
    for j in range(_DC):
        yc = y4_ref[0, j]
        nan = jnp.isnan(yc)
        ycc = jnp.where(nan, 0.0, yc)
        out1_ref[0, j * _L:(j + 1) * _L, :] = (
            base + ycc * w_y + jnp.where(nan, 1.0, 0.0) * delta)
        out2_ref[0, j * _L:(j + 1) * _L, :] = jnp.broadcast_to(
            space_ref[j], (_L, _DM))
        out3_ref[j] = jnp.full((1, _L), c * _DC + j, dtype=jnp.int32)


def kernel(x, y, t2v_w, t2v_b, local_table, vt_w, vt_b, space_table,
           given_table):
    batch, length, dy = y.shape
    x3 = x.reshape(batch, length, _DX)
    y4 = jnp.transpose(y, (0, 2, 1)).reshape(batch, dy, length, 1)
    vtb2 = vt_b.reshape(1, _DM)
    space3 = space_table.reshape(dy, 1, _DM)

    nc = dy // _DC
    grid = (batch, nc)
    out1, out2, out3 = pl.pallas_call(
        _tc_body,
        grid=grid,
        in_specs=[
            pl.BlockSpec((1, length, _DX), lambda b, c: (b, 0, 0)),
            pl.BlockSpec((1, _DC, length, 1), lambda b, c: (b, c, 0, 0)),
            pl.BlockSpec((_DX, _K), lambda b, c: (0, 0)),
            pl.BlockSpec((_DX, _K), lambda b, c: (0, 0)),
            pl.BlockSpec((length, _DM), lambda b, c: (0, 0)),
            pl.BlockSpec((_DX * _K + 1, _DM), lambda b, c: (0, 0)),
            pl.BlockSpec((1, _DM), lambda b, c: (0, 0)),
            pl.BlockSpec((_DC, 1, _DM), lambda b, c: (c, 0, 0)),
            pl.BlockSpec((2, _DM), lambda b, c: (0, 0)),
        ],
        out_specs=[
            pl.BlockSpec((1, _DC * length, _DM), lambda b, c: (b, c, 0)),
            pl.BlockSpec((1, _DC * length, _DM), lambda b, c: (b, c, 0)),
            pl.BlockSpec((_DC, 1, length), lambda b, c: (b * nc + c, 0, 0)),
        ],
        out_shape=[
            jax.ShapeDtypeStruct((batch, dy * length, _DM), jnp.float32),
            jax.ShapeDtypeStruct((batch, dy * length, _DM), jnp.float32),
            jax.ShapeDtypeStruct((batch * dy, 1, length), jnp.int32),
        ],
        scratch_shapes=[pltpu.VMEM((length, _DM), jnp.float32)],
        compiler_params=pltpu.CompilerParams(
            dimension_semantics=("arbitrary", "arbitrary")),
    )(x3, y4, t2v_w, t2v_b, local_table[:length], vt_w, vtb2, space3,
      given_table)

    return (out1, out2, out3.reshape(batch, dy * length))


# speedup vs baseline: 10.5044x; 1.0719x over previous
import jax
import jax.numpy as jnp
from jax import lax
from jax.experimental import pallas as pl
from jax.experimental.pallas import tpu as pltpu

_B, _L, _DY, _DX, _DM = 8, 512, 32, 6, 128
_K = 6


_DC = 16


def _tc_body(x_ref, y4_ref, t2vw_ref, t2vb_ref, local_ref, vtw_ref, vtb_ref,
             space_ref, given_ref, out1_ref, out2_ref, out3_ref, base_ref):
    c = pl.program_id(1)

    @pl.when(c == 0)
    def _compute_base():
        xs = x_ref[0]
        xs = jnp.where(jnp.isnan(xs), 0.0, xs)
        acc = local_ref[...] + vtb_ref[...] + given_ref[1:2, :]
        for i in range(_DX):
            aff = xs[:, i:i + 1] * t2vw_ref[i:i + 1, :] + t2vb_ref[i:i + 1, :]
            col = lax.broadcasted_iota(jnp.int32, aff.shape, 1)
            te_i = jnp.where(col == 0, aff, jnp.sin(aff))
            acc = acc + jnp.dot(te_i, vtw_ref[i * _K:(i + 1) * _K, :],
                                preferred_element_type=jnp.float32)
        base_ref[...] = acc

    w_y = vtw_ref[_DX * _K:_DX * _K + 1, :]
    delta = given_ref[0:1, :] - given_ref[1:2, :]
    base = base_ref[...]
    for j in range(_DC):
        yc = y4_ref[0, j]
        nan = jnp.isnan(yc)
        ycc = jnp.where(nan, 0.0, yc)
        out1_ref[0, j * _L:(j + 1) * _L, :] = (
            base + ycc * w_y + jnp.where(nan, 1.0, 0.0) * delta)
        out2_ref[0, j * _L:(j + 1) * _L, :] = jnp.broadcast_to(
            space_ref[j], (_L, _DM))
        out3_ref[j] = jnp.full((1, _L), c * _DC + j, dtype=jnp.int32)


def kernel(x, y, t2v_w, t2v_b, local_table, vt_w, vt_b, space_table,
           given_table):
    batch, length, dy = y.shape
    x3 = x.reshape(batch, length, _DX)
    y4 = jnp.transpose(y, (0, 2, 1)).reshape(batch, dy, length, 1)
    vtb2 = vt_b.reshape(1, _DM)
    space3 = space_table.reshape(dy, 1, _DM)

    nc = dy // _DC
    grid = (batch, nc)
    out1, out2, out3 = pl.pallas_call(
        _tc_body,
        grid=grid,
        in_specs=[
            pl.BlockSpec((1, length, _DX), lambda b, c: (b, 0, 0)),
            pl.BlockSpec((1, _DC, length, 1), lambda b, c: (b, c, 0, 0)),
            pl.BlockSpec((_DX, _K), lambda b, c: (0, 0)),
            pl.BlockSpec((_DX, _K), lambda b, c: (0, 0)),
            pl.BlockSpec((length, _DM), lambda b, c: (0, 0)),
            pl.BlockSpec((_DX * _K + 1, _DM), lambda b, c: (0, 0)),
            pl.BlockSpec((1, _DM), lambda b, c: (0, 0)),
            pl.BlockSpec((_DC, 1, _DM), lambda b, c: (c, 0, 0)),
            pl.BlockSpec((2, _DM), lambda b, c: (0, 0)),
        ],
        out_specs=[
            pl.BlockSpec((1, _DC * length, _DM), lambda b, c: (b, c, 0)),
            pl.BlockSpec((1, _DC * length, _DM), lambda b, c: (b, c, 0)),
            pl.BlockSpec((_DC, 1, length), lambda b, c: (b * nc + c, 0, 0)),
        ],
        out_shape=[
            jax.ShapeDtypeStruct((batch, dy * length, _DM), jnp.float32),
            jax.ShapeDtypeStruct((batch, dy * length, _DM), jnp.float32),
            jax.ShapeDtypeStruct((batch * dy, 1, length), jnp.int32),
        ],
        scratch_shapes=[pltpu.VMEM((length, _DM), jnp.float32)],
        compiler_params=pltpu.CompilerParams(
            dimension_semantics=("arbitrary", "arbitrary")),
    )(x3, y4, t2v_w, t2v_b, local_table[:length], vt_w, vtb2, space3,
      given_table)

    return (out1, out2, out3.reshape(batch, dy * length))


# speedup vs baseline: 13.4721x; 1.2825x over previous
import jax
import jax.numpy as jnp
from jax import lax
from jax.experimental import pallas as pl
from jax.experimental.pallas import tpu as pltpu

_B, _L, _DY, _DX, _DM = 8, 512, 32, 6, 128
_K = 6


_DC = 16

_S0 = 0.99999986216691
_S1 = -0.16666607728014005
_S2 = 0.008332732437814282
_S3 = -0.0001981669232761085
_S4 = 2.708326132222227e-06
_S5 = -2.069597015432612e-08
_INV_2PI = 0.15915494309189535
_2PI_HI = 6.28125
_2PI_LO = 1.9353071795864769e-03


def _fast_sin(a):
    k = jnp.round(a * _INV_2PI)
    r = a - k * _2PI_HI - k * _2PI_LO
    r2 = r * r
    return r * (_S0 + r2 * (_S1 + r2 * (_S2 + r2 * (
        _S3 + r2 * (_S4 + r2 * _S5)))))


def _tc_body(x_ref, y4_ref, t2vw_ref, t2vb_ref, local_ref, vtw_ref, vtb_ref,
             space_ref, given_ref, out1_ref, out2_ref, out3_ref, base_ref):
    c = pl.program_id(1)

    @pl.when(c == 0)
    def _compute_base():
        xs = x_ref[0]
        xs = jnp.where(jnp.isnan(xs), 0.0, xs)
        aff = xs * t2vw_ref[...] + t2vb_ref[...]
        col = lax.broadcasted_iota(jnp.int32, aff.shape, 1)
        te = jnp.where(col % _K == 0, aff, _fast_sin(aff))
        base_ref[...] = (local_ref[...] + vtb_ref[...] + given_ref[1:2, :]
                         + jnp.dot(te, vtw_ref[0:_DX * _K, :],
                                   preferred_element_type=jnp.float32))

    w_y = vtw_ref[_DX * _K:_DX * _K + 1, :]
    delta = given_ref[0:1, :] - given_ref[1:2, :]
    base = base_ref[...]
    for j in range(_DC):
        yc = y4_ref[0, j]
        nan = jnp.isnan(yc)
        ycc = jnp.where(nan, 0.0, yc)
        out1_ref[0, j * _L:(j + 1) * _L, :] = (
            base + ycc * w_y + jnp.where(nan, 1.0, 0.0) * delta)
        out2_ref[0, j * _L:(j + 1) * _L, :] = jnp.broadcast_to(
            space_ref[j], (_L, _DM))
        out3_ref[j] = jnp.full((1, _L), c * _DC + j, dtype=jnp.int32)


def kernel(x, y, t2v_w, t2v_b, local_table, vt_w, vt_b, space_table,
           given_table):
    batch, length, dy = y.shape
    x36 = jnp.repeat(x.reshape(batch, length, _DX), _K, axis=-1)
    wflat = t2v_w.reshape(1, _DX * _K)
    bflat = t2v_b.reshape(1, _DX * _K)
    y4 = jnp.transpose(y, (0, 2, 1)).reshape(batch, dy, length, 1)
    vtb2 = vt_b.reshape(1, _DM)
    space3 = space_table.reshape(dy, 1, _DM)

    nc = dy // _DC
    grid = (batch, nc)
    out1, out2, out3 = pl.pallas_call(
        _tc_body,
        grid=grid,
        in_specs=[
            pl.BlockSpec((1, length, _DX * _K), lambda b, c: (b, 0, 0)),
            pl.BlockSpec((1, _DC, length, 1), lambda b, c: (b, c, 0, 0)),
            pl.BlockSpec((1, _DX * _K), lambda b, c: (0, 0)),
            pl.BlockSpec((1, _DX * _K), lambda b, c: (0, 0)),
            pl.BlockSpec((length, _DM), lambda b, c: (0, 0)),
            pl.BlockSpec((_DX * _K + 1, _DM), lambda b, c: (0, 0)),
            pl.BlockSpec((1, _DM), lambda b, c: (0, 0)),
            pl.BlockSpec((_DC, 1, _DM), lambda b, c: (c, 0, 0)),
            pl.BlockSpec((2, _DM), lambda b, c: (0, 0)),
        ],
        out_specs=[
            pl.BlockSpec((1, _DC * length, _DM), lambda b, c: (b, c, 0)),
            pl.BlockSpec((1, _DC * length, _DM), lambda b, c: (b, c, 0)),
            pl.BlockSpec((_DC, 1, length), lambda b, c: (b * nc + c, 0, 0)),
        ],
        out_shape=[
            jax.ShapeDtypeStruct((batch, dy * length, _DM), jnp.float32),
            jax.ShapeDtypeStruct((batch, dy * length, _DM), jnp.float32),
            jax.ShapeDtypeStruct((batch * dy, 1, length), jnp.int32),
        ],
        scratch_shapes=[pltpu.VMEM((length, _DM), jnp.float32)],
        compiler_params=pltpu.CompilerParams(
            dimension_semantics=("arbitrary", "arbitrary")),
    )(x36, y4, wflat, bflat, local_table[:length], vt_w, vtb2, space3,
      given_table)

    return (out1, out2, out3.reshape(batch, dy * length))
